# Initial kernel scaffold; baseline (speedup 1.0000x reference)
import jax
import jax.numpy as jnp
from jax.experimental import pallas as pl


def kernel(a0, a1):
    raise NotImplementedError("write your pallas kernel here")


# speedup vs baseline: 1.2883x; 1.2883x over previous
import jax
import jax.numpy as jnp
from jax.experimental import pallas as pl


def _add_one_body(a0_ref, a1_ref, o0_ref, o1_ref):
    o0_ref[...] = a0_ref[...] + 1.0
    o1_ref[...] = a1_ref[...] + 1.0


def kernel(a0, a1):
    return pl.pallas_call(
        _add_one_body,
        out_shape=(
            jax.ShapeDtypeStruct((2,), jnp.float32),
            jax.ShapeDtypeStruct((4,), jnp.float32),
        ),
    )(a0, a1)
